# Initial kernel scaffold; baseline (speedup 1.0000x reference)
import jax
import jax.numpy as jnp
from jax.experimental import pallas as pl


def kernel(query_points, ref_points):
    raise NotImplementedError("write your pallas kernel here")


# speedup vs baseline: 3.4295x; 3.4295x over previous
import functools

import jax
import jax.numpy as jnp
from jax.experimental import pallas as pl

_QB = 128
_K = 8


def _knn_kernel(qa_ref, qb_ref, ra_ref, rb_ref, idx_ref, sum_ref, *, n_ref):
    qa = qa_ref[...]
    qx = qa[:, 0:1]
    qy = qa[:, 1:2]
    qz = qa[:, 2:3]
    q2 = qa[:, 3:4]
    rx = ra_ref[0:1, :]
    ry = ra_ref[1:2, :]
    rz = ra_ref[2:3, :]
    r2 = ra_ref[3:4, :]

    dot = jax.lax.dot_general(
        qb_ref[...], rb_ref[...], (((1,), (0,)), ((), ())),
        preferred_element_type=jnp.float32)
    d2r = (q2 - 2.0 * dot) + r2

    dx = qx - rx
    dy = qy - ry
    dz = qz - rz
    d2e = dx * dx + dy * dy + dz * dz

    iota = jax.lax.broadcasted_iota(jnp.int32, d2r.shape, 1)
    big = jnp.int32(n_ref)
    cols = []
    selmask = None
    for k in range(_K):
        m = jnp.min(d2r, axis=1, keepdims=True)
        sel = jnp.where(d2r == m, iota, big)
        idx = jnp.min(sel, axis=1, keepdims=True)
        cols.append(idx)
        hit = iota == idx
        selmask = hit if selmask is None else (selmask | hit)
        if k < _K - 1:
            d2r = jnp.where(hit, jnp.inf, d2r)
    idx_ref[...] = jnp.concatenate(cols, axis=1)

    me = jnp.min(jnp.where(selmask, d2e, jnp.inf), axis=1, keepdims=True)
    part = jnp.sum(jnp.sqrt(me), axis=0, keepdims=True)
    i = pl.program_id(0)

    @pl.when(i == 0)
    def _init():
        sum_ref[...] = part

    @pl.when(i != 0)
    def _acc():
        sum_ref[...] += part


def kernel(query_points, ref_points):
    n_query = query_points.shape[0]
    n_ref = ref_points.shape[0]
    f32 = jnp.float32

    q2 = jnp.sum(query_points * query_points, axis=1)
    r2 = jnp.sum(ref_points * ref_points, axis=1)

    q_aug = jnp.zeros((n_query, 8), f32)
    q_aug = q_aug.at[:, 0:3].set(query_points).at[:, 3].set(q2)
    r_aug = jnp.zeros((8, n_ref), f32)
    r_aug = r_aug.at[0:3, :].set(ref_points.T).at[3, :].set(r2)

    q_bf = jnp.zeros((n_query, 8), jnp.bfloat16)
    q_bf = q_bf.at[:, 0:3].set(query_points.astype(jnp.bfloat16))
    r_bf = jnp.zeros((8, n_ref), jnp.bfloat16)
    r_bf = r_bf.at[0:3, :].set(ref_points.T.astype(jnp.bfloat16))

    grid = n_query // _QB
    idx, ssum = pl.pallas_call(
        functools.partial(_knn_kernel, n_ref=n_ref),
        grid=(grid,),
        in_specs=[
            pl.BlockSpec((_QB, 8), lambda i: (i, 0)),
            pl.BlockSpec((_QB, 8), lambda i: (i, 0)),
            pl.BlockSpec((8, n_ref), lambda i: (0, 0)),
            pl.BlockSpec((8, n_ref), lambda i: (0, 0)),
        ],
        out_specs=[
            pl.BlockSpec((_QB, _K), lambda i: (i, 0)),
            pl.BlockSpec((1, 1), lambda i: (0, 0)),
        ],
        out_shape=[
            jax.ShapeDtypeStruct((n_query, _K), jnp.int32),
            jax.ShapeDtypeStruct((1, 1), jnp.float32),
        ],
    )(q_aug, q_bf, r_aug, r_bf)
    total = ssum[0, 0]
    loss = total * total / n_query / n_query
    return (loss, idx)


# speedup vs baseline: 3.6134x; 1.0536x over previous
import functools

import jax
import jax.numpy as jnp
from jax.experimental import pallas as pl

_QB = 256
_K = 8


def _knn_kernel(qa_ref, qb_ref, ra_ref, rb_ref, idx_ref, sum_ref, *, n_ref):
    qa = qa_ref[...]
    qx = qa[:, 0:1]
    qy = qa[:, 1:2]
    qz = qa[:, 2:3]
    q2 = qa[:, 3:4]
    rx = ra_ref[0:1, :]
    ry = ra_ref[1:2, :]
    rz = ra_ref[2:3, :]
    r2 = ra_ref[3:4, :]

    dot = jax.lax.dot_general(
        qb_ref[...], rb_ref[...], (((1,), (0,)), ((), ())),
        preferred_element_type=jnp.float32)
    d2r = (q2 - 2.0 * dot) + r2

    dx = qx - rx
    dy = qy - ry
    dz = qz - rz
    d2e = dx * dx + dy * dy + dz * dz

    iota = jax.lax.broadcasted_iota(jnp.int32, d2r.shape, 1)
    big = jnp.int32(n_ref)
    cols = []
    for k in range(_K):
        m = jnp.min(d2r, axis=1, keepdims=True)
        sel = jnp.where(d2r == m, iota, big)
        idx = jnp.min(sel, axis=1, keepdims=True)
        cols.append(idx)
        d2r = jnp.where(iota == idx, jnp.inf, d2r)
    idx_ref[...] = jnp.concatenate(cols, axis=1)

    me = jnp.min(jnp.where(d2r == jnp.inf, d2e, jnp.inf), axis=1, keepdims=True)
    part = jnp.sum(jnp.sqrt(me), axis=0, keepdims=True)
    i = pl.program_id(0)

    @pl.when(i == 0)
    def _init():
        sum_ref[...] = part

    @pl.when(i != 0)
    def _acc():
        sum_ref[...] += part


def kernel(query_points, ref_points):
    n_query = query_points.shape[0]
    n_ref = ref_points.shape[0]
    f32 = jnp.float32

    q2 = jnp.sum(query_points * query_points, axis=1)
    r2 = jnp.sum(ref_points * ref_points, axis=1)

    q_aug = jnp.zeros((n_query, 8), f32)
    q_aug = q_aug.at[:, 0:3].set(query_points).at[:, 3].set(q2)
    r_aug = jnp.zeros((8, n_ref), f32)
    r_aug = r_aug.at[0:3, :].set(ref_points.T).at[3, :].set(r2)

    q_bf = jnp.zeros((n_query, 8), jnp.bfloat16)
    q_bf = q_bf.at[:, 0:3].set(query_points.astype(jnp.bfloat16))
    r_bf = jnp.zeros((8, n_ref), jnp.bfloat16)
    r_bf = r_bf.at[0:3, :].set(ref_points.T.astype(jnp.bfloat16))

    grid = n_query // _QB
    idx, ssum = pl.pallas_call(
        functools.partial(_knn_kernel, n_ref=n_ref),
        grid=(grid,),
        in_specs=[
            pl.BlockSpec((_QB, 8), lambda i: (i, 0)),
            pl.BlockSpec((_QB, 8), lambda i: (i, 0)),
            pl.BlockSpec((8, n_ref), lambda i: (0, 0)),
            pl.BlockSpec((8, n_ref), lambda i: (0, 0)),
        ],
        out_specs=[
            pl.BlockSpec((_QB, _K), lambda i: (i, 0)),
            pl.BlockSpec((1, 1), lambda i: (0, 0)),
        ],
        out_shape=[
            jax.ShapeDtypeStruct((n_query, _K), jnp.int32),
            jax.ShapeDtypeStruct((1, 1), jnp.float32),
        ],
    )(q_aug, q_bf, r_aug, r_bf)
    total = ssum[0, 0]
    loss = total * total / n_query / n_query
    return (loss, idx)
